# baseline (speedup 1.0000x reference)
import jax
import jax.numpy as jnp
from jax.experimental import pallas as pl
from jax.experimental.pallas import tpu as pltpu

N = 10000
BM = 400
BK = 1024
NK = (N + BK - 1) // BK
NPAD = NK * BK


def _s0_kernel(x_ref, w0_ref, o_ref):
    o_ref[...] = jnp.dot(
        x_ref[...].astype(jnp.bfloat16), w0_ref[...].astype(jnp.bfloat16),
        preferred_element_type=jnp.float32).astype(jnp.bfloat16)


def _phase_a_kernel(adj_ref, s0_ref, b0_ref, w1_ref, s1_ref, pout_ref,
                    s1v_ref):
    i = pl.program_id(0)

    @pl.when(i == 0)
    def _init():
        s1v_ref[...] = jnp.zeros_like(s1v_ref)

    a = adj_ref[...].astype(jnp.bfloat16)
    pout_ref[...] = jnp.dot(a, s1v_ref[...], preferred_element_type=jnp.float32)
    g = jnp.dot(a, s0_ref[...], preferred_element_type=jnp.float32)
    h = jnp.maximum(g + b0_ref[...], 0.0).astype(jnp.bfloat16)
    s1_i = jnp.dot(h, w1_ref[...].astype(jnp.bfloat16),
                   preferred_element_type=jnp.float32).astype(jnp.bfloat16)
    s1_ref[...] = s1_i
    s1v_ref[pl.ds(i * BM, BM), :] = s1_i


def _phase_b_kernel(adj_ref, s1_ref, pout_ref, b1_ref, o_ref, acc_ref):
    i = pl.program_id(0)
    t = pl.program_id(1)
    nt = pl.num_programs(1)
    jstart = (BM * i) // BK

    @pl.when(t == 0)
    def _init():
        acc_ref[...] = pout_ref[...]

    @pl.when(jstart + t <= NK - 1)
    def _accum():
        j = jstart + t
        col = j * BK + jax.lax.broadcasted_iota(jnp.int32, (1, BK), 1)
        keep = (col >= BM * i) & (col < N)
        a = jnp.where(keep, adj_ref[...], 0.0).astype(jnp.bfloat16)
        acc_ref[...] += jnp.dot(a, s1_ref[...],
                                preferred_element_type=jnp.float32)

    @pl.when(t == nt - 1)
    def _fin():
        z = acc_ref[...] + b1_ref[...]
        m = jnp.max(z, axis=-1, keepdims=True)
        z = z - m
        lse = jnp.log(jnp.sum(jnp.exp(z), axis=-1, keepdims=True))
        o_ref[...] = z - lse


@jax.jit
def kernel(x, adj, W0, b0, W1, b1):
    nfeat = x.shape[1]
    nhid = W0.shape[1]
    ncls = W1.shape[1]

    s0 = pl.pallas_call(
        _s0_kernel,
        grid=(5,),
        in_specs=[
            pl.BlockSpec((N // 5, nfeat), lambda i: (i, 0)),
            pl.BlockSpec((nfeat, nhid), lambda i: (0, 0)),
        ],
        out_specs=pl.BlockSpec((N // 5, nhid), lambda i: (i, 0)),
        out_shape=jax.ShapeDtypeStruct((N, nhid), jnp.bfloat16),
    )(x, W0)

    s1, pout = pl.pallas_call(
        _phase_a_kernel,
        grid=(N // BM,),
        in_specs=[
            pl.BlockSpec((BM, N), lambda i: (i, 0)),
            pl.BlockSpec((N, nhid), lambda i: (0, 0)),
            pl.BlockSpec((1, nhid), lambda i: (0, 0)),
            pl.BlockSpec((nhid, ncls), lambda i: (0, 0)),
        ],
        out_specs=[
            pl.BlockSpec((BM, ncls), lambda i: (i, 0)),
            pl.BlockSpec((BM, ncls), lambda i: (i, 0)),
        ],
        out_shape=[
            jax.ShapeDtypeStruct((N, ncls), jnp.bfloat16),
            jax.ShapeDtypeStruct((N, ncls), jnp.float32),
        ],
        scratch_shapes=[pltpu.VMEM((N, ncls), jnp.bfloat16)],
        compiler_params=pltpu.CompilerParams(
            dimension_semantics=("arbitrary",)),
    )(adj, s0, b0.reshape(1, nhid), W1)

    s1p = jnp.pad(s1, ((0, NPAD - N), (0, 0)))

    def _adj_col(i, t):
        return (i, jnp.minimum((BM * i) // BK + t, NK - 1))

    out = pl.pallas_call(
        _phase_b_kernel,
        grid=(N // BM, NK),
        in_specs=[
            pl.BlockSpec((BM, BK), _adj_col),
            pl.BlockSpec((BK, ncls), lambda i, t: (_adj_col(i, t)[1], 0)),
            pl.BlockSpec((BM, ncls), lambda i, t: (i, 0)),
            pl.BlockSpec((1, ncls), lambda i, t: (0, 0)),
        ],
        out_specs=pl.BlockSpec((BM, ncls), lambda i, t: (i, 0)),
        out_shape=jax.ShapeDtypeStruct((N, ncls), jnp.float32),
        scratch_shapes=[pltpu.VMEM((BM, ncls), jnp.float32)],
        compiler_params=pltpu.CompilerParams(
            dimension_semantics=("arbitrary", "arbitrary")),
    )(adj, s1p, pout, b1.reshape(1, ncls))

    return out
